# Initial kernel scaffold; baseline (speedup 1.0000x reference)
import jax
import jax.numpy as jnp
from jax.experimental import pallas as pl


def kernel(prediction, target, mask):
    raise NotImplementedError("write your pallas kernel here")


# speedup vs baseline: 59.7087x; 59.7087x over previous
import functools

import jax
import jax.numpy as jnp
from jax import lax
from jax.experimental import pallas as pl
from jax.experimental.pallas import tpu as pltpu
from jax.experimental.pallas import tpu_sc as plsc

NC = 2
NS = 16
L = 16
NW = NC * NS

N_TOTAL = 32 * 384 * 384
PER_W = N_TOTAL // NW
CHUNK = 16384
NCHUNK = PER_W // CHUNK


def _sc_partials(p, t, m):
    mesh = plsc.VectorSubcoreMesh(core_axis_name="c", subcore_axis_name="s")

    @functools.partial(
        pl.kernel,
        mesh=mesh,
        out_type=(
            jax.ShapeDtypeStruct((NW, L), jnp.float32),
            jax.ShapeDtypeStruct((NW, L), jnp.int32),
        ),
        scratch_types=[
            pltpu.VMEM((CHUNK,), jnp.float32),
            pltpu.VMEM((CHUNK,), jnp.float32),
            pltpu.VMEM((CHUNK,), jnp.int32),
            pltpu.VMEM((L,), jnp.float32),
            pltpu.VMEM((L,), jnp.int32),
        ],
    )
    def k(p_hbm, t_hbm, m_hbm, num_hbm, cnt_hbm, p_v, t_v, m_v, num_v, cnt_v):
        wid = lax.axis_index("s") * NC + lax.axis_index("c")
        base = wid * PER_W

        def chunk_body(ci, carry):
            acc, cnt = carry
            off = base + ci * CHUNK
            pltpu.sync_copy(p_hbm.at[pl.ds(off, CHUNK)], p_v)
            pltpu.sync_copy(t_hbm.at[pl.ds(off, CHUNK)], t_v)
            pltpu.sync_copy(m_hbm.at[pl.ds(off, CHUNK)], m_v)

            def vec_body(i, c2):
                a2, n2 = c2
                pv = p_v[pl.ds(i * L, L)]
                tv = t_v[pl.ds(i * L, L)]
                mv = m_v[pl.ds(i * L, L)]
                ad = jnp.abs(pv - tv)
                a2 = a2 + jnp.where(mv != 0, ad, 0.0)
                n2 = n2 + mv
                return a2, n2

            return lax.fori_loop(0, CHUNK // L, vec_body, (acc, cnt))

        acc0 = jnp.zeros((L,), jnp.float32)
        cnt0 = jnp.zeros((L,), jnp.int32)
        acc, cnt = lax.fori_loop(0, NCHUNK, chunk_body, (acc0, cnt0))
        num_v[...] = acc
        cnt_v[...] = cnt
        pltpu.sync_copy(num_v, num_hbm.at[wid])
        pltpu.sync_copy(cnt_v, cnt_hbm.at[wid])

    return k(p, t, m)


def _finish_body(num_ref, cnt_ref, out_ref):
    s = jnp.sum(num_ref[...])
    c = jnp.sum(cnt_ref[...].astype(jnp.float32))
    out_ref[...] = (s / (2.0 * c)).reshape(1, 1)


def kernel(prediction, target, mask):
    p = prediction.reshape(-1)
    t = target.reshape(-1)
    m = mask.reshape(-1)
    num, cnt = _sc_partials(p, t, m)
    out = pl.pallas_call(
        _finish_body,
        out_shape=jax.ShapeDtypeStruct((1, 1), jnp.float32),
    )(num, cnt)
    return out[0, 0]


# speedup vs baseline: 77.6185x; 1.3000x over previous
import functools

import jax
import jax.numpy as jnp
from jax import lax
from jax.experimental import pallas as pl
from jax.experimental.pallas import tpu as pltpu
from jax.experimental.pallas import tpu_sc as plsc

NC = 2
NS = 16
L = 16
NW = NC * NS

N_TOTAL = 32 * 384 * 384
PER_W = N_TOTAL // NW
CHUNK = 18432
NCHUNK = PER_W // CHUNK
NVEC = CHUNK // L
UNIT = 4


def _sc_partials(p, t, m):
    mesh = plsc.VectorSubcoreMesh(core_axis_name="c", subcore_axis_name="s")

    @functools.partial(
        pl.kernel,
        mesh=mesh,
        out_type=(
            jax.ShapeDtypeStruct((NW, L), jnp.float32),
            jax.ShapeDtypeStruct((NW, L), jnp.int32),
        ),
        scratch_types=[
            pltpu.VMEM((2, CHUNK), jnp.float32),
            pltpu.VMEM((2, CHUNK), jnp.float32),
            pltpu.VMEM((2, CHUNK), jnp.int32),
            pltpu.VMEM((L,), jnp.float32),
            pltpu.VMEM((L,), jnp.int32),
            pltpu.SemaphoreType.DMA,
            pltpu.SemaphoreType.DMA,
        ],
    )
    def k(p_hbm, t_hbm, m_hbm, num_hbm, cnt_hbm,
          p_v, t_v, m_v, num_v, cnt_v, sem0, sem1):
        wid = lax.axis_index("s") * NC + lax.axis_index("c")
        base = wid * PER_W
        sems = (sem0, sem1)

        def issue(ci):
            slot = ci % 2
            off = base + ci * CHUNK
            sl = pl.ds(off, CHUNK)
            return (
                pltpu.async_copy(p_hbm.at[sl], p_v.at[slot], sems[slot]),
                pltpu.async_copy(t_hbm.at[sl], t_v.at[slot], sems[slot]),
                pltpu.async_copy(m_hbm.at[sl], m_v.at[slot], sems[slot]),
            )

        def compute(slot, acc, cnt):
            pr, tr, mr = p_v.at[slot], t_v.at[slot], m_v.at[slot]
            zero = jnp.zeros((L,), jnp.float32)
            zeroi = jnp.zeros((L,), jnp.int32)
            carry0 = (acc, zero, zero, zero, cnt, zeroi, zeroi, zeroi)

            @plsc.parallel_loop(0, NVEC, step=UNIT, unroll=2, carry=carry0)
            def body(i, c):
                a = list(c[:UNIT])
                n = list(c[UNIT:])
                for u in range(UNIT):
                    sl = pl.ds((i + u) * L, L)
                    ad = jnp.abs(pr[sl] - tr[sl])
                    mv = mr[sl]
                    a[u] = a[u] + jnp.where(mv != 0, ad, 0.0)
                    n[u] = n[u] + mv
                return tuple(a) + tuple(n)

            c = body
            return (c[0] + c[1]) + (c[2] + c[3]), (c[4] + c[5]) + (c[6] + c[7])

        acc = jnp.zeros((L,), jnp.float32)
        cnt = jnp.zeros((L,), jnp.int32)
        handles = {0: issue(0)}
        for ci in range(NCHUNK):
            if ci + 1 < NCHUNK:
                handles[ci + 1] = issue(ci + 1)
            for h in handles.pop(ci):
                h.wait()
            acc, cnt = compute(ci % 2, acc, cnt)
        num_v[...] = acc
        cnt_v[...] = cnt
        pltpu.sync_copy(num_v, num_hbm.at[wid])
        pltpu.sync_copy(cnt_v, cnt_hbm.at[wid])

    return k(p, t, m)


def _finish_body(num_ref, cnt_ref, out_ref):
    s = jnp.sum(num_ref[...])
    c = jnp.sum(cnt_ref[...].astype(jnp.float32))
    out_ref[...] = (s / (2.0 * c)).reshape(1, 1)


def kernel(prediction, target, mask):
    p = prediction.reshape(-1)
    t = target.reshape(-1)
    m = mask.reshape(-1)
    num, cnt = _sc_partials(p, t, m)
    out = pl.pallas_call(
        _finish_body,
        out_shape=jax.ShapeDtypeStruct((1, 1), jnp.float32),
    )(num, cnt)
    return out[0, 0]


# speedup vs baseline: 184.1678x; 2.3727x over previous
import functools

import jax
import jax.numpy as jnp
from jax import lax
from jax.experimental import pallas as pl
from jax.experimental.pallas import tpu as pltpu
from jax.experimental.pallas import tpu_sc as plsc

NC = 2
NS = 16
L = 16
NW = NC * NS

B, H, W = 32, 384, 384
ROWS = 48
NCHUNK = H // ROWS
VPR = W // L
NVEC = ROWS * VPR
UNIT = 4


def _sc_partials(p, t, m):
    mesh = plsc.VectorSubcoreMesh(core_axis_name="c", subcore_axis_name="s")

    @functools.partial(
        pl.kernel,
        mesh=mesh,
        out_type=(
            jax.ShapeDtypeStruct((NW, L), jnp.float32),
            jax.ShapeDtypeStruct((NW, L), jnp.int32),
        ),
        scratch_types=[
            pltpu.VMEM((2, ROWS, W), jnp.float32),
            pltpu.VMEM((2, ROWS, W), jnp.float32),
            pltpu.VMEM((2, ROWS, W), jnp.int32),
            pltpu.VMEM((L,), jnp.float32),
            pltpu.VMEM((L,), jnp.int32),
            pltpu.SemaphoreType.DMA,
            pltpu.SemaphoreType.DMA,
        ],
    )
    def k(p_hbm, t_hbm, m_hbm, num_hbm, cnt_hbm,
          p_v, t_v, m_v, num_v, cnt_v, sem0, sem1):
        wid = lax.axis_index("s") * NC + lax.axis_index("c")
        sems = (sem0, sem1)

        def issue(ci):
            slot = ci % 2
            sl = pl.ds(ci * ROWS, ROWS)
            return (
                pltpu.async_copy(p_hbm.at[wid, sl], p_v.at[slot], sems[slot]),
                pltpu.async_copy(t_hbm.at[wid, sl], t_v.at[slot], sems[slot]),
                pltpu.async_copy(m_hbm.at[wid, sl], m_v.at[slot], sems[slot]),
            )

        def compute(slot, acc, cnt):
            pr, tr, mr = p_v.at[slot], t_v.at[slot], m_v.at[slot]
            zero = jnp.zeros((L,), jnp.float32)
            zeroi = jnp.zeros((L,), jnp.int32)
            carry0 = (acc, zero, zero, zero, cnt, zeroi, zeroi, zeroi)

            @plsc.parallel_loop(0, NVEC, step=UNIT, unroll=2, carry=carry0)
            def body(i, c):
                a = list(c[:UNIT])
                n = list(c[UNIT:])
                r = i // VPR
                c0 = (i - r * VPR) * L
                for u in range(UNIT):
                    sl = pl.ds(c0 + u * L, L)
                    ad = jnp.abs(pr[r, sl] - tr[r, sl])
                    mv = mr[r, sl]
                    a[u] = a[u] + jnp.where(mv != 0, ad, 0.0)
                    n[u] = n[u] + mv
                return tuple(a) + tuple(n)

            c = body
            return (c[0] + c[1]) + (c[2] + c[3]), (c[4] + c[5]) + (c[6] + c[7])

        acc = jnp.zeros((L,), jnp.float32)
        cnt = jnp.zeros((L,), jnp.int32)
        handles = {0: issue(0)}
        for ci in range(NCHUNK):
            if ci + 1 < NCHUNK:
                handles[ci + 1] = issue(ci + 1)
            for h in handles.pop(ci):
                h.wait()
            acc, cnt = compute(ci % 2, acc, cnt)
        num_v[...] = acc
        cnt_v[...] = cnt
        pltpu.sync_copy(num_v, num_hbm.at[wid])
        pltpu.sync_copy(cnt_v, cnt_hbm.at[wid])

    return k(p, t, m)


def _finish_body(num_ref, cnt_ref, out_ref):
    s = jnp.sum(num_ref[...])
    c = jnp.sum(cnt_ref[...].astype(jnp.float32))
    out_ref[...] = (s / (2.0 * c)).reshape(1, 1)


def kernel(prediction, target, mask):
    num, cnt = _sc_partials(prediction, target, mask)
    out = pl.pallas_call(
        _finish_body,
        out_shape=jax.ShapeDtypeStruct((1, 1), jnp.float32),
    )(num, cnt)
    return out[0, 0]
